# baseline (speedup 1.0000x reference)
import dataclasses
import functools

import jax
import jax.numpy as jnp
from jax import lax
from jax.experimental import pallas as pl
from jax.experimental.pallas import tpu as pltpu
from jax.experimental.pallas import tpu_sc as plsc

D = 768
LANES = 16
DC = D // LANES
CHUNK = 32
TB = 8
EPS = 1e-5


def _rsqrt(x):
    i = lax.bitcast_convert_type(x, jnp.int32)
    i = jnp.int32(0x5F3759DF) - lax.shift_right_logical(i, 1)
    y = lax.bitcast_convert_type(i, jnp.float32)
    for _ in range(3):
        y = y * (1.5 - 0.5 * x * y * y)
    return y


def _sc_embed(ids, segs, word_emb, pos_emb, seg_emb, gamma, beta, *, seq_len):
    tokens = ids.shape[0]
    info = plsc.get_sparse_core_info()
    nw = info.num_cores * info.num_subcores
    per_w = tokens // nw
    nchunk = per_w // CHUNK
    assert nchunk % 2 == 0

    mesh = plsc.VectorSubcoreMesh(core_axis_name="c", subcore_axis_name="s")

    cp = pltpu.CompilerParams()
    if "needs_layout_passes" in pltpu.CompilerParams.__dataclass_fields__:
        cp = dataclasses.replace(cp, needs_layout_passes=False)

    @functools.partial(
        pl.kernel,
        mesh=mesh,
        compiler_params=cp,
        out_type=jax.ShapeDtypeStruct((tokens, D), jnp.float32),
        scratch_types=[
            pltpu.VMEM((per_w,), jnp.int32),
            pltpu.VMEM((per_w,), jnp.int32),
            pltpu.VMEM((CHUNK, D), jnp.float32),
            pltpu.VMEM((CHUNK, D), jnp.float32),
            pltpu.VMEM((CHUNK, D), jnp.float32),
            pltpu.VMEM((CHUNK, D), jnp.float32),
            pltpu.VMEM((2, D), jnp.float32),
            pltpu.VMEM((2, D), jnp.float32),
            pltpu.SemaphoreType.DMA,
            pltpu.SemaphoreType.DMA,
            pltpu.SemaphoreType.DMA,
            pltpu.SemaphoreType.DMA,
            pltpu.SemaphoreType.DMA,
            pltpu.SemaphoreType.DMA,
        ],
    )
    def k(ids_hbm, segs_hbm, word_hbm, pos_hbm, segemb_hbm, gamma_hbm,
          beta_hbm, out_hbm, ids_v, segs_v, rows0, rows1, posb0, posb1,
          se_v, gb_v, gs0, gs1, ps0, ps1, os0, os1):
        c = lax.axis_index("c")
        s = lax.axis_index("s")
        wid = s * info.num_cores + c
        tok_base = wid * per_w
        pos_base = lax.rem(tok_base, seq_len)

        rows = (rows0, rows1)
        posb = (posb0, posb1)
        gsem = (gs0, gs1)
        psem = (ps0, ps1)
        osem = (os0, os1)

        pltpu.sync_copy(ids_hbm.at[pl.ds(tok_base, per_w)], ids_v)
        pltpu.sync_copy(segs_hbm.at[pl.ds(tok_base, per_w)], segs_v)
        pltpu.sync_copy(segemb_hbm, se_v)
        pltpu.sync_copy(gamma_hbm, gb_v.at[0])
        pltpu.sync_copy(beta_hbm, gb_v.at[1])

        @pl.loop(0, DC)
        def _dseg(j):
            sl = pl.ds(j * LANES, LANES)
            se_v[1, sl] = se_v[1, sl] - se_v[0, sl]

        def in_copies(ci, b):
            tok0 = ci * CHUNK
            return (
                pltpu.make_async_copy(
                    word_hbm.at[ids_v.at[pl.ds(tok0, CHUNK)]], rows[b],
                    gsem[b]),
                pltpu.make_async_copy(
                    pos_hbm.at[pl.ds(pos_base + tok0, CHUNK)], posb[b],
                    psem[b]),
            )

        def out_copy(ci, b):
            return pltpu.make_async_copy(
                rows[b], out_hbm.at[pl.ds(tok_base + ci * CHUNK, CHUNK)],
                osem[b])

        def start_in(ci, b):
            for cp_ in in_copies(ci, b):
                cp_.start()

        def wait_in(ci, b):
            for cp_ in in_copies(ci, b):
                cp_.wait()

        def compute_chunk(ci, b):
            rb = rows[b]
            pb = posb[b]
            tok0 = ci * CHUNK

            @pl.loop(0, CHUNK // LANES)
            def _half(h):
                base = h * LANES
                segf = segs_v[pl.ds(tok0 + base, LANES)].astype(jnp.float32)
                for blk in range(LANES // TB):
                    tb = base + blk * TB
                    sf = [segf[blk * TB + i] for i in range(TB)]

                    zero = jnp.zeros((LANES,), jnp.float32)

                    def p1(j, carry):
                        sl = pl.ds(j * LANES, LANES)
                        se0 = se_v[0, sl]
                        dse = se_v[1, sl]
                        out = []
                        for tt in range(TB):
                            x = rb[tb + tt, sl] + pb[tb + tt, sl]
                            x = x + (se0 + sf[tt] * dse)
                            rb[tb + tt, sl] = x
                            out.append(carry[2 * tt] + x)
                            out.append(carry[2 * tt + 1] + x * x)
                        return tuple(out)

                    acc = plsc.parallel_loop(
                        0, DC, unroll=3, carry=(zero,) * (2 * TB))(p1)

                    means = []
                    invs = []
                    for tt in range(TB):
                        m = jnp.sum(acc[2 * tt]) * (1.0 / D)
                        v = jnp.sum(acc[2 * tt + 1]) * (1.0 / D) - m * m
                        means.append(m)
                        invs.append(_rsqrt(v + EPS))

                    @plsc.parallel_loop(0, DC, unroll=4)
                    def _p2(j):
                        sl = pl.ds(j * LANES, LANES)
                        g = gb_v[0, sl]
                        bt = gb_v[1, sl]
                        for tt in range(TB):
                            a = invs[tt] * g
                            cv = bt - means[tt] * a
                            rb[tb + tt, sl] = rb[tb + tt, sl] * a + cv

        start_in(0, 0)

        @pl.loop(0, nchunk, step=2)
        def _pipe(cc):

            @pl.when(cc >= 2)
            def _():
                out_copy(cc - 1, 1).wait()
            start_in(cc + 1, 1)
            wait_in(cc, 0)
            compute_chunk(cc, 0)
            out_copy(cc, 0).start()

            @pl.when(cc + 2 < nchunk)
            def _():
                out_copy(cc, 0).wait()
                start_in(cc + 2, 0)
            wait_in(cc + 1, 1)
            compute_chunk(cc + 1, 1)
            out_copy(cc + 1, 1).start()

        out_copy(nchunk - 2, 0).wait()
        out_copy(nchunk - 1, 1).wait()

    return k(ids, segs, word_emb, pos_emb, seg_emb, gamma, beta)


def kernel(input_ids, segment_ids, word_emb, pos_emb, seg_emb, gamma, beta):
    b, s = input_ids.shape
    ids = input_ids.reshape(-1).astype(jnp.int32)
    segs = segment_ids.reshape(-1).astype(jnp.int32)
    out = _sc_embed(ids, segs, word_emb, pos_emb, seg_emb, gamma, beta,
                    seq_len=s)
    return out.reshape(b, s, D)


# speedup vs baseline: 1.0429x; 1.0429x over previous
import dataclasses
import functools

import jax
import jax.numpy as jnp
from jax import lax
from jax.experimental import pallas as pl
from jax.experimental.pallas import tpu as pltpu
from jax.experimental.pallas import tpu_sc as plsc

D = 768
LANES = 16
DC = D // LANES
CHUNK = 32
TB = 16
EPS = 1e-5


def _rsqrt(x):
    i = lax.bitcast_convert_type(x, jnp.int32)
    i = jnp.int32(0x5F3759DF) - lax.shift_right_logical(i, 1)
    y = lax.bitcast_convert_type(i, jnp.float32)
    for _ in range(3):
        y = y * (1.5 - 0.5 * x * y * y)
    return y


def _sc_embed(ids, segs, word_emb, pos_emb, seg_emb, gamma, beta, *, seq_len):
    tokens = ids.shape[0]
    info = plsc.get_sparse_core_info()
    nw = info.num_cores * info.num_subcores
    per_w = tokens // nw
    nchunk = per_w // CHUNK
    assert nchunk % 2 == 0

    mesh = plsc.VectorSubcoreMesh(core_axis_name="c", subcore_axis_name="s")

    cp = pltpu.CompilerParams()
    if "needs_layout_passes" in pltpu.CompilerParams.__dataclass_fields__:
        cp = dataclasses.replace(cp, needs_layout_passes=False)

    @functools.partial(
        pl.kernel,
        mesh=mesh,
        compiler_params=cp,
        out_type=jax.ShapeDtypeStruct((tokens, D), jnp.float32),
        scratch_types=[
            pltpu.VMEM((per_w,), jnp.int32),
            pltpu.VMEM((per_w,), jnp.int32),
            pltpu.VMEM((CHUNK, D), jnp.float32),
            pltpu.VMEM((CHUNK, D), jnp.float32),
            pltpu.VMEM((CHUNK, D), jnp.float32),
            pltpu.VMEM((CHUNK, D), jnp.float32),
            pltpu.VMEM((2, D), jnp.float32),
            pltpu.VMEM((2, D), jnp.float32),
            pltpu.SemaphoreType.DMA,
            pltpu.SemaphoreType.DMA,
            pltpu.SemaphoreType.DMA,
            pltpu.SemaphoreType.DMA,
            pltpu.SemaphoreType.DMA,
            pltpu.SemaphoreType.DMA,
        ],
    )
    def k(ids_hbm, segs_hbm, word_hbm, pos_hbm, segemb_hbm, gamma_hbm,
          beta_hbm, out_hbm, ids_v, segs_v, rows0, rows1, posb0, posb1,
          se_v, gb_v, gs0, gs1, ps0, ps1, os0, os1):
        c = lax.axis_index("c")
        s = lax.axis_index("s")
        wid = s * info.num_cores + c
        tok_base = wid * per_w
        pos_base = lax.rem(tok_base, seq_len)

        rows = (rows0, rows1)
        posb = (posb0, posb1)
        gsem = (gs0, gs1)
        psem = (ps0, ps1)
        osem = (os0, os1)

        pltpu.sync_copy(ids_hbm.at[pl.ds(tok_base, per_w)], ids_v)
        pltpu.sync_copy(segs_hbm.at[pl.ds(tok_base, per_w)], segs_v)
        pltpu.sync_copy(segemb_hbm, se_v)
        pltpu.sync_copy(gamma_hbm, gb_v.at[0])
        pltpu.sync_copy(beta_hbm, gb_v.at[1])

        @pl.loop(0, DC)
        def _dseg(j):
            sl = pl.ds(j * LANES, LANES)
            se_v[1, sl] = se_v[1, sl] - se_v[0, sl]

        def in_copies(ci, b):
            tok0 = ci * CHUNK
            return (
                pltpu.make_async_copy(
                    word_hbm.at[ids_v.at[pl.ds(tok0, CHUNK)]], rows[b],
                    gsem[b]),
                pltpu.make_async_copy(
                    pos_hbm.at[pl.ds(pos_base + tok0, CHUNK)], posb[b],
                    psem[b]),
            )

        def out_copy(ci, b):
            return pltpu.make_async_copy(
                rows[b], out_hbm.at[pl.ds(tok_base + ci * CHUNK, CHUNK)],
                osem[b])

        def start_in(ci, b):
            for cp_ in in_copies(ci, b):
                cp_.start()

        def wait_in(ci, b):
            for cp_ in in_copies(ci, b):
                cp_.wait()

        def compute_chunk(ci, b):
            rb = rows[b]
            pb = posb[b]
            tok0 = ci * CHUNK

            @pl.loop(0, CHUNK // LANES)
            def _half(h):
                base = h * LANES
                segf = segs_v[pl.ds(tok0 + base, LANES)].astype(jnp.float32)
                for blk in range(LANES // TB):
                    tb = base + blk * TB
                    sf = [segf[blk * TB + i] for i in range(TB)]

                    zero = jnp.zeros((LANES,), jnp.float32)

                    def p1(j, carry):
                        sl = pl.ds(j * LANES, LANES)
                        se0 = se_v[0, sl]
                        dse = se_v[1, sl]
                        out = []
                        for tt in range(TB):
                            x = rb[tb + tt, sl] + pb[tb + tt, sl]
                            x = x + (se0 + sf[tt] * dse)
                            rb[tb + tt, sl] = x
                            out.append(carry[2 * tt] + x)
                            out.append(carry[2 * tt + 1] + x * x)
                        return tuple(out)

                    acc = plsc.parallel_loop(
                        0, DC, unroll=1, carry=(zero,) * (2 * TB))(p1)

                    means = []
                    invs = []
                    for tt in range(TB):
                        m = jnp.sum(acc[2 * tt]) * (1.0 / D)
                        v = jnp.sum(acc[2 * tt + 1]) * (1.0 / D) - m * m
                        means.append(m)
                        invs.append(_rsqrt(v + EPS))

                    @plsc.parallel_loop(0, DC, unroll=2)
                    def _p2(j):
                        sl = pl.ds(j * LANES, LANES)
                        g = gb_v[0, sl]
                        bt = gb_v[1, sl]
                        for tt in range(TB):
                            a = invs[tt] * g
                            cv = bt - means[tt] * a
                            rb[tb + tt, sl] = rb[tb + tt, sl] * a + cv

        start_in(0, 0)

        @pl.loop(0, nchunk, step=2)
        def _pipe(cc):

            @pl.when(cc >= 2)
            def _():
                out_copy(cc - 1, 1).wait()
            start_in(cc + 1, 1)
            wait_in(cc, 0)
            compute_chunk(cc, 0)
            out_copy(cc, 0).start()

            @pl.when(cc + 2 < nchunk)
            def _():
                out_copy(cc, 0).wait()
                start_in(cc + 2, 0)
            wait_in(cc + 1, 1)
            compute_chunk(cc + 1, 1)
            out_copy(cc + 1, 1).start()

        out_copy(nchunk - 2, 0).wait()
        out_copy(nchunk - 1, 1).wait()

    return k(ids, segs, word_emb, pos_emb, seg_emb, gamma, beta)


def kernel(input_ids, segment_ids, word_emb, pos_emb, seg_emb, gamma, beta):
    b, s = input_ids.shape
    ids = input_ids.reshape(-1).astype(jnp.int32)
    segs = segment_ids.reshape(-1).astype(jnp.int32)
    out = _sc_embed(ids, segs, word_emb, pos_emb, seg_emb, gamma, beta,
                    seq_len=s)
    return out.reshape(b, s, D)


# speedup vs baseline: 1.5826x; 1.5175x over previous
import dataclasses
import functools

import jax
import jax.numpy as jnp
from jax import lax
from jax.experimental import pallas as pl
from jax.experimental.pallas import tpu as pltpu
from jax.experimental.pallas import tpu_sc as plsc

D = 768
LANES = 16
DC = D // LANES
CHUNK = 32
TB = 8
EPS = 1e-5


def _rsqrt(x):
    i = lax.bitcast_convert_type(x, jnp.int32)
    i = jnp.int32(0x5F3759DF) - lax.shift_right_logical(i, 1)
    y = lax.bitcast_convert_type(i, jnp.float32)
    for _ in range(3):
        y = y * (1.5 - 0.5 * x * y * y)
    return y


def _sc_embed(ids, segs, word_emb, pos_emb, seg_emb, gamma, beta, *, seq_len):
    tokens = ids.shape[0]
    info = plsc.get_sparse_core_info()
    nw = info.num_cores * info.num_subcores
    per_w = tokens // nw
    nchunk = per_w // CHUNK
    assert nchunk % 2 == 0

    mesh = plsc.VectorSubcoreMesh(core_axis_name="c", subcore_axis_name="s")

    cp = pltpu.CompilerParams()
    if "needs_layout_passes" in pltpu.CompilerParams.__dataclass_fields__:
        cp = dataclasses.replace(cp, needs_layout_passes=False)

    @functools.partial(
        pl.kernel,
        mesh=mesh,
        compiler_params=cp,
        out_type=jax.ShapeDtypeStruct((tokens, D), jnp.float32),
        scratch_types=[
            pltpu.VMEM((per_w,), jnp.int32),
            pltpu.VMEM((per_w,), jnp.int32),
            pltpu.VMEM((CHUNK, D), jnp.float32),
            pltpu.VMEM((CHUNK, D), jnp.float32),
            pltpu.VMEM((CHUNK, D), jnp.float32),
            pltpu.VMEM((CHUNK, D), jnp.float32),
            pltpu.VMEM((2, D), jnp.float32),
            pltpu.VMEM((2, D), jnp.float32),
            pltpu.SemaphoreType.DMA,
            pltpu.SemaphoreType.DMA,
            pltpu.SemaphoreType.DMA,
            pltpu.SemaphoreType.DMA,
            pltpu.SemaphoreType.DMA,
            pltpu.SemaphoreType.DMA,
        ],
    )
    def k(ids_hbm, segs_hbm, word_hbm, pos_hbm, segemb_hbm, gamma_hbm,
          beta_hbm, out_hbm, ids_v, segs_v, rows0, rows1, posb0, posb1,
          se_v, gb_v, gs0, gs1, ps0, ps1, os0, os1):
        c = lax.axis_index("c")
        s = lax.axis_index("s")
        wid = s * info.num_cores + c
        tok_base = wid * per_w
        pos_base = lax.rem(tok_base, seq_len)

        rows = (rows0, rows1)
        posb = (posb0, posb1)
        gsem = (gs0, gs1)
        psem = (ps0, ps1)
        osem = (os0, os1)

        ids_cp = pltpu.make_async_copy(
            ids_hbm.at[pl.ds(tok_base, per_w)], ids_v, os0)
        segs_cp = pltpu.make_async_copy(
            segs_hbm.at[pl.ds(tok_base, per_w)], segs_v, os1)
        se_cp = pltpu.make_async_copy(segemb_hbm, se_v, gs1)
        g_cp = pltpu.make_async_copy(gamma_hbm, gb_v.at[0], ps1)
        b_cp = pltpu.make_async_copy(beta_hbm, gb_v.at[1], ps1)
        for cp_ in (ids_cp, segs_cp, se_cp, g_cp, b_cp):
            cp_.start()

        def in_copies(ci, b):
            tok0 = ci * CHUNK
            return (
                pltpu.make_async_copy(
                    word_hbm.at[ids_v.at[pl.ds(tok0, CHUNK)]], rows[b],
                    gsem[b]),
                pltpu.make_async_copy(
                    pos_hbm.at[pl.ds(pos_base + tok0, CHUNK)], posb[b],
                    psem[b]),
            )

        def out_copy(ci, b):
            return pltpu.make_async_copy(
                rows[b], out_hbm.at[pl.ds(tok_base + ci * CHUNK, CHUNK)],
                osem[b])

        def start_in(ci, b):
            for cp_ in in_copies(ci, b):
                cp_.start()

        def wait_in(ci, b):
            for cp_ in in_copies(ci, b):
                cp_.wait()

        def compute_chunk(ci, b):
            rb = rows[b]
            pb = posb[b]
            tok0 = ci * CHUNK

            @pl.loop(0, CHUNK // LANES)
            def _half(h):
                base = h * LANES
                segf = segs_v[pl.ds(tok0 + base, LANES)].astype(jnp.float32)
                for blk in range(LANES // TB):
                    tb = base + blk * TB
                    sf = [segf[blk * TB + i] for i in range(TB)]

                    zero = jnp.zeros((LANES,), jnp.float32)

                    def p1(j, carry):
                        sl = pl.ds(j * LANES, LANES)
                        se0 = se_v[0, sl]
                        dse = se_v[1, sl]
                        out = []
                        for tt in range(TB):
                            x = rb[tb + tt, sl] + pb[tb + tt, sl]
                            x = x + (se0 + sf[tt] * dse)
                            rb[tb + tt, sl] = x
                            out.append(carry[2 * tt] + x)
                            out.append(carry[2 * tt + 1] + x * x)
                        return tuple(out)

                    acc = plsc.parallel_loop(
                        0, DC, unroll=2, carry=(zero,) * (2 * TB))(p1)

                    means = []
                    invs = []
                    for tt in range(TB):
                        m = jnp.sum(acc[2 * tt]) * (1.0 / D)
                        v = jnp.sum(acc[2 * tt + 1]) * (1.0 / D) - m * m
                        means.append(m)
                        invs.append(_rsqrt(v + EPS))

                    @plsc.parallel_loop(0, DC, unroll=2)
                    def _p2(j):
                        sl = pl.ds(j * LANES, LANES)
                        g = gb_v[0, sl]
                        bt = gb_v[1, sl]
                        for tt in range(TB):
                            a = invs[tt] * g
                            cv = bt - means[tt] * a
                            rb[tb + tt, sl] = rb[tb + tt, sl] * a + cv

        ids_cp.wait()
        start_in(0, 0)
        for cp_ in (segs_cp, se_cp, g_cp, b_cp):
            cp_.wait()

        @plsc.parallel_loop(0, DC)
        def _dseg(j):
            sl = pl.ds(j * LANES, LANES)
            se_v[1, sl] = se_v[1, sl] - se_v[0, sl]

        @pl.loop(0, nchunk, step=2)
        def _pipe(cc):

            @pl.when(cc >= 2)
            def _():
                out_copy(cc - 1, 1).wait()
            start_in(cc + 1, 1)
            wait_in(cc, 0)
            compute_chunk(cc, 0)
            out_copy(cc, 0).start()

            @pl.when(cc + 2 < nchunk)
            def _():
                out_copy(cc, 0).wait()
                start_in(cc + 2, 0)
            wait_in(cc + 1, 1)
            compute_chunk(cc + 1, 1)
            out_copy(cc + 1, 1).start()

        out_copy(nchunk - 2, 0).wait()
        out_copy(nchunk - 1, 1).wait()

    return k(ids, segs, word_emb, pos_emb, seg_emb, gamma, beta)


def kernel(input_ids, segment_ids, word_emb, pos_emb, seg_emb, gamma, beta):
    b, s = input_ids.shape
    ids = input_ids.reshape(-1).astype(jnp.int32)
    segs = segment_ids.reshape(-1).astype(jnp.int32)
    out = _sc_embed(ids, segs, word_emb, pos_emb, seg_emb, gamma, beta,
                    seq_len=s)
    return out.reshape(b, s, D)
